# baseline (speedup 1.0000x reference)
import jax
import jax.numpy as jnp
from jax.experimental import pallas as pl
from jax.experimental.pallas import tpu as pltpu

D_MODEL = 128
BATCH = 32
BUFROWS = 10000
NBUF = 4
CHUNKS = [1000, 1000, 2000, 4000, 8000] + [10000] * 8 + [4000]
assert sum(CHUNKS) == 100000
_OFFS = [sum(CHUNKS[:i]) for i in range(len(CHUNKS))]


def _attn_read_kernel(q_ref, k_hbm, v_hbm, o_ref, kbuf, vbuf,
                      ksem0, ksem1, vsem0, vsem1):

    def copies(hbm, buf, sem0, sem1, c):
        b = c % NBUF
        n = CHUNKS[c]
        h = n // 2
        return (
            pltpu.make_async_copy(
                hbm.at[pl.ds(_OFFS[c], h)], buf.at[b, pl.ds(0, h)], sem0.at[b]),
            pltpu.make_async_copy(
                hbm.at[pl.ds(_OFFS[c] + h, h)], buf.at[b, pl.ds(h, h)], sem1.at[b]),
        )

    def start(hbm, buf, sem0, sem1, c):
        a, bb = copies(hbm, buf, sem0, sem1, c)
        a.start()
        bb.start()

    def wait(hbm, buf, sem0, sem1, c):
        a, bb = copies(hbm, buf, sem0, sem1, c)
        a.wait()
        bb.wait()

    nchunks = len(CHUNKS)
    for c in range(NBUF):
        start(k_hbm, kbuf, ksem0, ksem1, c)
        start(v_hbm, vbuf, vsem0, vsem1, c)

    q = q_ref[...]
    m = jnp.full((BATCH, 1), -jnp.inf, dtype=jnp.float32)
    l = jnp.zeros((BATCH, 1), dtype=jnp.float32)
    acc = jnp.zeros((BATCH, D_MODEL), dtype=jnp.float32)

    for c in range(nchunks):
        b = c % NBUF
        n = CHUNKS[c]
        wait(k_hbm, kbuf, ksem0, ksem1, c)
        s = jax.lax.dot_general(
            q, kbuf[b, 0:n], (((1,), (1,)), ((), ())),
            preferred_element_type=jnp.float32,
        ) * (1.0 / (D_MODEL ** 0.5))
        m_new = jnp.maximum(m, jnp.max(s, axis=1, keepdims=True))
        alpha = jnp.exp(m - m_new)
        p = jnp.exp(s - m_new)
        l = alpha * l + jnp.sum(p, axis=1, keepdims=True)
        m = m_new
        wait(v_hbm, vbuf, vsem0, vsem1, c)
        pv = jax.lax.dot_general(
            p, vbuf[b, 0:n], (((1,), (0,)), ((), ())),
            preferred_element_type=jnp.float32,
        )
        acc = acc * alpha + pv
        if c + NBUF < nchunks:
            start(k_hbm, kbuf, ksem0, ksem1, c + NBUF)
            start(v_hbm, vbuf, vsem0, vsem1, c + NBUF)

    o_ref[...] = acc / l


def kernel(hidden_state, keys, values):
    return pl.pallas_call(
        _attn_read_kernel,
        grid=(1,),
        in_specs=[
            pl.BlockSpec((BATCH, D_MODEL), lambda i: (0, 0)),
            pl.BlockSpec(memory_space=pl.ANY),
            pl.BlockSpec(memory_space=pl.ANY),
        ],
        out_specs=pl.BlockSpec((BATCH, D_MODEL), lambda i: (0, 0)),
        out_shape=jax.ShapeDtypeStruct((BATCH, D_MODEL), jnp.float32),
        scratch_shapes=[
            pltpu.VMEM((NBUF, BUFROWS, D_MODEL), jnp.float32),
            pltpu.VMEM((NBUF, BUFROWS, D_MODEL), jnp.float32),
            pltpu.SemaphoreType.DMA((NBUF,)),
            pltpu.SemaphoreType.DMA((NBUF,)),
            pltpu.SemaphoreType.DMA((NBUF,)),
            pltpu.SemaphoreType.DMA((NBUF,)),
        ],
    )(hidden_state, keys, values)
